# baseline (speedup 1.0000x reference)
import jax
import jax.numpy as jnp
from jax.experimental import pallas as pl
from jax.experimental.pallas import tpu as pltpu

CLASS_NUM = 80
CONF_THRE = 0.2
NMS_THRE = 0.45

N = 5000
NPAD = 5120
ROWS = 8
COLS = 640
RB = 8


def _rmax(x):
    return jnp.max(x, axis=(0, 1), keepdims=True)


def _rmin(x):
    return jnp.min(x, axis=(0, 1), keepdims=True)


def _dm_kernel(pred_ref, zone_ref,
               y1o, x1o, y2o, x2o, inzko, sco, clso, cyo, cxo, keepo,
               x1r, y1r, x2r, y2r, arear, iotar):
    f32 = jnp.float32
    cx = pred_ref[0]
    cy = pred_ref[1]
    w = pred_ref[2]
    h = pred_ref[3]
    obj = pred_ref[4]
    x1r[...] = cx - w / 2
    y1r[...] = cy - h / 2
    x2r[...] = cx + w / 2
    y2r[...] = cy + h / 2
    arear[...] = (jnp.clip(x2r[...] - x1r[...], 0.0)
                  * jnp.clip(y2r[...] - y1r[...], 0.0))
    iotar[...] = (jax.lax.broadcasted_iota(jnp.int32, (ROWS, COLS), 0) * COLS
                  + jax.lax.broadcasted_iota(jnp.int32, (ROWS, COLS), 1))

    def cbody(k, carry):
        best, bk = carry
        v = pred_ref[5 + k]
        gt = v > best
        return jnp.where(gt, v, best), jnp.where(gt, k, bk)

    best0 = pred_ref[5]
    bk0 = jnp.zeros((ROWS, COLS), jnp.int32)
    class_conf, class_pred = jax.lax.fori_loop(1, CLASS_NUM, cbody, (best0, bk0))

    snms = obj * class_conf
    valid = snms >= CONF_THRE
    s0 = jnp.where(valid, snms, f32(-1.0))
    m0 = jnp.max(s0)

    def nms_cond(carry):
        _, m = carry
        return m >= CONF_THRE

    def nms_body(carry):
        s, _ = carry
        iota = iotar[...]
        x1 = x1r[...]
        y1 = y1r[...]
        x2 = x2r[...]
        y2 = y2r[...]
        area = arear[...]
        ninf = f32(-jnp.inf)

        sels = []
        valids = []
        coords = []
        s_cur = s
        for _k in range(RB):
            mkv = _rmax(s_cur)
            ikv = _rmin(jnp.where(s_cur == mkv, iota, jnp.int32(NPAD)))
            sel_k = iota == ikv
            s_cur = jnp.where(sel_k, f32(-1.0), s_cur)
            x1s = _rmax(jnp.where(sel_k, x1, ninf))
            y1s = _rmax(jnp.where(sel_k, y1, ninf))
            x2s = _rmax(jnp.where(sel_k, x2, ninf))
            y2s = _rmax(jnp.where(sel_k, y2, ninf))
            sels.append(sel_k)
            valids.append(mkv >= CONF_THRE)
            coords.append((x1s, y1s, x2s, y2s))

        rows = []
        for _k in range(RB):
            x1s, y1s, x2s, y2s = coords[_k]
            areas = jnp.clip(x2s - x1s, 0.0) * jnp.clip(y2s - y1s, 0.0)
            ltx = jnp.maximum(x1s, x1)
            lty = jnp.maximum(y1s, y1)
            rbx = jnp.minimum(x2s, x2)
            rby = jnp.minimum(y2s, y2)
            iw = jnp.clip(rbx - ltx, 0.0)
            ih = jnp.clip(rby - lty, 0.0)
            inter = iw * ih
            union = areas + area - inter
            iou = inter / jnp.maximum(union, f32(1e-9))
            rows.append(iou > NMS_THRE)

        supp = jnp.zeros((ROWS, COLS), jnp.bool_)
        selacc = jnp.zeros((ROWS, COLS), jnp.bool_)
        for _k in range(RB):
            is_supp = jnp.any(sels[_k] & supp, axis=(0, 1), keepdims=True)
            acc_k = jnp.logical_and(jnp.logical_not(is_supp), valids[_k])
            supp = supp | (rows[_k] & acc_k)
            selacc = selacc | (sels[_k] & acc_k)

        s2 = jnp.where(selacc, f32(-2.0), jnp.where(supp, f32(-1.0), s))
        return s2, jnp.max(s2)

    sf, _ = jax.lax.while_loop(nms_cond, nms_body, (s0, m0))
    keepb = sf == f32(-2.0)
    mk = jnp.where(keepb, f32(1.0), f32(0.0))

    x1 = x1r[...]
    y1 = y1r[...]
    x2 = x2r[...]
    y2 = y2r[...]
    px = (x1 + x2) / 2
    py = (y1 + y2) / 2

    parity = jnp.zeros((ROWS, COLS), jnp.bool_)
    for k in range(8):
        xi = zone_ref[k, 0]
        yi = zone_ref[k, 1]
        xj = zone_ref[(k - 1) % 8, 0]
        yj = zone_ref[(k - 1) % 8, 1]
        gyi = yi > py
        gyj = yj > py
        gx = (xj - xi) * (py - yi) / (yj - yi) + xi
        parity = parity ^ ((gyi != gyj) & (gx > px))

    y1o[...] = y1 * mk
    x1o[...] = x1 * mk
    y2o[...] = y2 * mk
    x2o[...] = x2 * mk
    inzko[...] = (parity & keepb).astype(jnp.int32)
    sco[...] = jnp.maximum(obj, class_conf) * mk
    clso[...] = jnp.where(keepb, class_pred, -1)
    cyo[...] = py * mk
    cxo[...] = px * mk
    keepo[...] = keepb.astype(jnp.int32)


def kernel(prediction, zone):
    p = prediction[0]
    pT = jnp.pad(jnp.transpose(p), ((0, 0), (0, NPAD - N)))
    pp = pT.reshape(85, ROWS, COLS)

    f32 = jnp.float32
    outs = pl.pallas_call(
        _dm_kernel,
        out_shape=[
            jax.ShapeDtypeStruct((ROWS, COLS), f32),
            jax.ShapeDtypeStruct((ROWS, COLS), f32),
            jax.ShapeDtypeStruct((ROWS, COLS), f32),
            jax.ShapeDtypeStruct((ROWS, COLS), f32),
            jax.ShapeDtypeStruct((ROWS, COLS), jnp.int32),
            jax.ShapeDtypeStruct((ROWS, COLS), f32),
            jax.ShapeDtypeStruct((ROWS, COLS), jnp.int32),
            jax.ShapeDtypeStruct((ROWS, COLS), f32),
            jax.ShapeDtypeStruct((ROWS, COLS), f32),
            jax.ShapeDtypeStruct((ROWS, COLS), jnp.int32),
        ],
        scratch_shapes=[
            pltpu.VMEM((ROWS, COLS), f32),
            pltpu.VMEM((ROWS, COLS), f32),
            pltpu.VMEM((ROWS, COLS), f32),
            pltpu.VMEM((ROWS, COLS), f32),
            pltpu.VMEM((ROWS, COLS), f32),
            pltpu.VMEM((ROWS, COLS), jnp.int32),
        ],
    )(pp, zone)

    y1m, x1m, y2m, x2m, inzk, sc, cls_o, cym, cxm, keep = [
        o.reshape(NPAD)[:N] for o in outs
    ]
    boxes_yxyx = jnp.stack([y1m, x1m, y2m, x2m], axis=1)
    centers_yx = jnp.stack([cym, cxm], axis=1)
    return (boxes_yxyx,
            inzk.astype(jnp.bool_),
            sc,
            cls_o,
            centers_yx,
            keep.astype(jnp.bool_))


# speedup vs baseline: 1.2734x; 1.2734x over previous
import jax
import jax.numpy as jnp
from jax.experimental import pallas as pl
from jax.experimental.pallas import tpu as pltpu

CLASS_NUM = 80
CONF_THRE = 0.2
NMS_THRE = 0.45

N = 5000
NPAD = 5120
ROWS = 8
COLS = 640
RB = 8


def _rmax(x):
    return jnp.max(x, axis=(0, 1), keepdims=True)


def _rmin(x):
    return jnp.min(x, axis=(0, 1), keepdims=True)


def _dm_kernel(pred_ref, pred4_ref, zone_ref,
               y1o, x1o, y2o, x2o, inzko, sco, clso, cyo, cxo, keepo,
               x1r, y1r, x2r, y2r, arear, iotar):
    f32 = jnp.float32
    cx = pred_ref[0]
    cy = pred_ref[1]
    w = pred_ref[2]
    h = pred_ref[3]
    obj = pred_ref[4]
    x1r[...] = cx - w / 2
    y1r[...] = cy - h / 2
    x2r[...] = cx + w / 2
    y2r[...] = cy + h / 2
    arear[...] = (jnp.clip(x2r[...] - x1r[...], 0.0)
                  * jnp.clip(y2r[...] - y1r[...], 0.0))
    iotar[...] = (jax.lax.broadcasted_iota(jnp.int32, (ROWS, COLS), 0) * COLS
                  + jax.lax.broadcasted_iota(jnp.int32, (ROWS, COLS), 1))

    def cbody(k, carry):
        best, bk = carry
        v = pred_ref[5 + k]
        gt = v > best
        return jnp.where(gt, v, best), jnp.where(gt, k, bk)

    best0 = pred_ref[5]
    bk0 = jnp.zeros((ROWS, COLS), jnp.int32)
    class_conf, class_pred = jax.lax.fori_loop(1, CLASS_NUM, cbody, (best0, bk0))

    snms = obj * class_conf
    valid = snms >= CONF_THRE
    s0 = jnp.where(valid, snms, f32(-1.0))
    m0 = jnp.max(s0)

    def nms_cond(carry):
        _, m = carry
        return m >= CONF_THRE

    def nms_body(carry):
        s, _ = carry
        iota = iotar[...]
        x1 = x1r[...]
        y1 = y1r[...]
        x2 = x2r[...]
        y2 = y2r[...]
        area = arear[...]
        ninf = f32(-jnp.inf)

        sels = []
        valids = []
        coords = []
        s_cur = s
        for _k in range(RB):
            mkv = _rmax(s_cur)
            ikv = _rmin(jnp.where(s_cur == mkv, iota, jnp.int32(NPAD)))
            sel_k = iota == ikv
            s_cur = jnp.where(sel_k, f32(-1.0), s_cur)
            i_k = ikv[0, 0]
            cxs = pred4_ref[0, i_k]
            cys = pred4_ref[1, i_k]
            ws = pred4_ref[2, i_k]
            hs = pred4_ref[3, i_k]
            x1s = cxs - ws / 2
            y1s = cys - hs / 2
            x2s = cxs + ws / 2
            y2s = cys + hs / 2
            areas = jnp.clip(x2s - x1s, 0.0) * jnp.clip(y2s - y1s, 0.0)
            sels.append(sel_k)
            valids.append(mkv[0, 0] >= CONF_THRE)
            coords.append((x1s, y1s, x2s, y2s, areas))

        accs = []
        for _k in range(RB):
            x1k, y1k, x2k, y2k, ak = coords[_k]
            ok = valids[_k]
            for _j in range(_k):
                x1j, y1j, x2j, y2j, aj = coords[_j]
                ltx = jnp.maximum(x1j, x1k)
                lty = jnp.maximum(y1j, y1k)
                rbx = jnp.minimum(x2j, x2k)
                rby = jnp.minimum(y2j, y2k)
                inter = jnp.clip(rbx - ltx, 0.0) * jnp.clip(rby - lty, 0.0)
                union = aj + ak - inter
                iou = inter / jnp.maximum(union, f32(1e-9))
                ok = ok & jnp.logical_not(accs[_j] & (iou > NMS_THRE))
            accs.append(ok)

        supp = jnp.zeros((ROWS, COLS), jnp.bool_)
        selacc = jnp.zeros((ROWS, COLS), jnp.bool_)
        for _k in range(RB):
            x1s, y1s, x2s, y2s, areas = coords[_k]
            ltx = jnp.maximum(x1s, x1)
            lty = jnp.maximum(y1s, y1)
            rbx = jnp.minimum(x2s, x2)
            rby = jnp.minimum(y2s, y2)
            iw = jnp.clip(rbx - ltx, 0.0)
            ih = jnp.clip(rby - lty, 0.0)
            inter = iw * ih
            union = areas + area - inter
            iou = inter / jnp.maximum(union, f32(1e-9))
            supp = supp | ((iou > NMS_THRE) & accs[_k])
            selacc = selacc | (sels[_k] & accs[_k])

        s2 = jnp.where(selacc, f32(-2.0), jnp.where(supp, f32(-1.0), s))
        return s2, jnp.max(s2)

    sf, _ = jax.lax.while_loop(nms_cond, nms_body, (s0, m0))
    keepb = sf == f32(-2.0)
    mk = jnp.where(keepb, f32(1.0), f32(0.0))

    x1 = x1r[...]
    y1 = y1r[...]
    x2 = x2r[...]
    y2 = y2r[...]
    px = (x1 + x2) / 2
    py = (y1 + y2) / 2

    parity = jnp.zeros((ROWS, COLS), jnp.bool_)
    for k in range(8):
        xi = zone_ref[k, 0]
        yi = zone_ref[k, 1]
        xj = zone_ref[(k - 1) % 8, 0]
        yj = zone_ref[(k - 1) % 8, 1]
        gyi = yi > py
        gyj = yj > py
        gx = (xj - xi) * (py - yi) / (yj - yi) + xi
        parity = parity ^ ((gyi != gyj) & (gx > px))

    y1o[...] = y1 * mk
    x1o[...] = x1 * mk
    y2o[...] = y2 * mk
    x2o[...] = x2 * mk
    inzko[...] = (parity & keepb).astype(jnp.int32)
    sco[...] = jnp.maximum(obj, class_conf) * mk
    clso[...] = jnp.where(keepb, class_pred, -1)
    cyo[...] = py * mk
    cxo[...] = px * mk
    keepo[...] = keepb.astype(jnp.int32)


def kernel(prediction, zone):
    p = prediction[0]
    pT = jnp.pad(jnp.transpose(p), ((0, 0), (0, NPAD - N)))
    pp = pT.reshape(85, ROWS, COLS)
    pred4 = pT[:4]

    f32 = jnp.float32
    outs = pl.pallas_call(
        _dm_kernel,
        in_specs=[
            pl.BlockSpec(memory_space=pltpu.VMEM),
            pl.BlockSpec(memory_space=pltpu.SMEM),
            pl.BlockSpec(memory_space=pltpu.SMEM),
        ],
        out_shape=[
            jax.ShapeDtypeStruct((ROWS, COLS), f32),
            jax.ShapeDtypeStruct((ROWS, COLS), f32),
            jax.ShapeDtypeStruct((ROWS, COLS), f32),
            jax.ShapeDtypeStruct((ROWS, COLS), f32),
            jax.ShapeDtypeStruct((ROWS, COLS), jnp.int32),
            jax.ShapeDtypeStruct((ROWS, COLS), f32),
            jax.ShapeDtypeStruct((ROWS, COLS), jnp.int32),
            jax.ShapeDtypeStruct((ROWS, COLS), f32),
            jax.ShapeDtypeStruct((ROWS, COLS), f32),
            jax.ShapeDtypeStruct((ROWS, COLS), jnp.int32),
        ],
        scratch_shapes=[
            pltpu.VMEM((ROWS, COLS), f32),
            pltpu.VMEM((ROWS, COLS), f32),
            pltpu.VMEM((ROWS, COLS), f32),
            pltpu.VMEM((ROWS, COLS), f32),
            pltpu.VMEM((ROWS, COLS), f32),
            pltpu.VMEM((ROWS, COLS), jnp.int32),
        ],
    )(pp, pred4, zone)

    y1m, x1m, y2m, x2m, inzk, sc, cls_o, cym, cxm, keep = [
        o.reshape(NPAD)[:N] for o in outs
    ]
    boxes_yxyx = jnp.stack([y1m, x1m, y2m, x2m], axis=1)
    centers_yx = jnp.stack([cym, cxm], axis=1)
    return (boxes_yxyx,
            inzk.astype(jnp.bool_),
            sc,
            cls_o,
            centers_yx,
            keep.astype(jnp.bool_))
